# baseline (speedup 1.0000x reference)
import functools

import jax
import jax.numpy as jnp
from jax import lax
from jax.experimental import pallas as pl
from jax.experimental.pallas import tpu as pltpu
from jax.experimental.pallas import tpu_sc as plsc

N_FIELDS = 26
VOCAB = 100000
DIM = 64
BATCH = 16384

NC = 2
NS = 16
LANES = 16
NW = NC * NS
BW = BATCH // NW
GB = 128
G = BW // GB
NBUF = 3


def _feature_sum_call():
    mesh = plsc.VectorSubcoreMesh(core_axis_name="c", subcore_axis_name="s")

    @functools.partial(
        pl.kernel,
        mesh=mesh,
        out_type=jax.ShapeDtypeStruct((BATCH, DIM), jnp.float32),
        compiler_params=pltpu.CompilerParams(use_tc_tiling_on_sc=False),
        scratch_types=[
            pltpu.VMEM((N_FIELDS, BW), jnp.int32),
            pltpu.VMEM((N_FIELDS, BW), jnp.int32),
            pltpu.VMEM((GB, DIM), jnp.float32),
            pltpu.VMEM((GB, DIM), jnp.float32),
            pltpu.VMEM((GB, DIM), jnp.float32),
            pltpu.VMEM((GB, DIM), jnp.float32),
            pltpu.SemaphoreType.DMA,
            pltpu.SemaphoreType.DMA,
            pltpu.SemaphoreType.DMA,
        ],
    )
    def k(xw_hbm, tab_hbm, out_hbm, xv, idx, b0, b1, b2, acc, s0, s1, s2):
        bufs = (b0, b1, b2)
        sems = (s0, s1, s2)
        wid = lax.axis_index("s") * NC + lax.axis_index("c")
        base = wid * BW

        pltpu.sync_copy(xw_hbm.at[wid], xv)

        for f in range(N_FIELDS):
            def build(t, _, f=f):
                v = xv[f, pl.ds(t * LANES, LANES)] + f * VOCAB
                idx[f, pl.ds(t * LANES, LANES)] = v
                return 0
            lax.fori_loop(0, BW // LANES, build, 0)

        def issue(s):
            g, f = divmod(s, N_FIELDS)
            return pltpu.async_copy(
                tab_hbm.at[idx.at[f, pl.ds(g * GB, GB)]],
                bufs[s % NBUF], sems[s % NBUF])

        def accum(s):
            buf = bufs[s % NBUF]
            f = s % N_FIELDS
            if f == 0:
                def bd(r, _):
                    for c in range(DIM // LANES):
                        sl = pl.ds(c * LANES, LANES)
                        acc[r, sl] = buf[r, sl]
                    return 0
            else:
                def bd(r, _):
                    for c in range(DIM // LANES):
                        sl = pl.ds(c * LANES, LANES)
                        plsc.addupdate(acc.at[r, sl], buf[r, sl])
                    return 0
            lax.fori_loop(0, GB, bd, 0)

        total = G * N_FIELDS
        pending = {}
        for s in range(min(NBUF - 1, total)):
            pending[s] = issue(s)
        for s in range(total):
            nxt = s + NBUF - 1
            if nxt < total:
                pending[nxt] = issue(nxt)
            pending.pop(s).wait()
            accum(s)
            if s % N_FIELDS == N_FIELDS - 1:
                g = s // N_FIELDS
                pltpu.sync_copy(acc, out_hbm.at[pl.ds(base + g * GB, GB), :])

    return k


def kernel(x, tables):
    xw = x.reshape(NW, BW, N_FIELDS).transpose(0, 2, 1)
    tab = tables.reshape(N_FIELDS * VOCAB, DIM)
    return _feature_sum_call()(xw, tab)
